# baseline (speedup 1.0000x reference)
import functools

import jax
import jax.numpy as jnp
from jax import lax
from jax.experimental import pallas as pl
from jax.experimental.pallas import tpu as pltpu
from jax.experimental.pallas import tpu_sc as plsc

NEG_SLOPE = 0.2
NC = 2
NS = 16
NW = NC * NS
CH = 96
LANES = 16
DEN_CH = 512


def _prep_body(x_ref, w_ref, q_ref, k_ref, xw_ref, sq_ref, sk_ref):
    xb = x_ref[...]
    o = jnp.dot(xb, w_ref[0], preferred_element_type=jnp.float32)
    xw_ref[0] = o
    sq_ref[0] = jnp.dot(o, q_ref[...], preferred_element_type=jnp.float32)
    sk_ref[0] = jnp.dot(o, k_ref[...], preferred_element_type=jnp.float32)


def _prep(x, w, q, k, bn):
    n, c = x.shape
    r = w.shape[0]
    grid = (r, n // bn)
    return pl.pallas_call(
        _prep_body,
        grid=grid,
        in_specs=[
            pl.BlockSpec((bn, c), lambda ri, i: (i, 0)),
            pl.BlockSpec((1, c, w.shape[2]), lambda ri, i: (ri, 0, 0)),
            pl.BlockSpec(q.shape, lambda ri, i: (0, 0)),
            pl.BlockSpec(k.shape, lambda ri, i: (0, 0)),
        ],
        out_specs=[
            pl.BlockSpec((1, bn, w.shape[2]), lambda ri, i: (ri, i, 0)),
            pl.BlockSpec((1, bn, 1), lambda ri, i: (ri, i, 0)),
            pl.BlockSpec((1, bn, 1), lambda ri, i: (ri, i, 0)),
        ],
        out_shape=[
            jax.ShapeDtypeStruct((r, n, w.shape[2]), jnp.float32),
            jax.ShapeDtypeStruct((r, n, 1), jnp.float32),
            jax.ShapeDtypeStruct((r, n, 1), jnp.float32),
        ],
    )(x, w, q, k)


def _eprep_body(dst_ref, src_ref, typ_ref, attr_ref, le1_ref, e1_ref,
                le2_ref, e2_ref, ed_ref, ae1_ref, ae2_ref, *, n):
    t = typ_ref[0]
    ed_ref[0, :, 0, :] = t * n + dst_ref[0]
    ed_ref[0, :, 1, :] = t * n + src_ref[0]
    ed_ref[0, :, 2, :] = dst_ref[0]
    c1 = jnp.sum(jnp.dot(le1_ref[...], e1_ref[...],
                         preferred_element_type=jnp.float32))
    c2 = jnp.sum(jnp.dot(le2_ref[...], e2_ref[...],
                         preferred_element_type=jnp.float32))
    a = attr_ref[0]
    ae1_ref[0, :, 0, :] = a * c1
    ae2_ref[0, :, 0, :] = a * c2


def _eprep(dst2, src2, typ2, attr2, le1, e1, le2, e2, n):
    g, rows, cols = dst2.shape
    grid = (g,)
    blk = pl.BlockSpec((1, rows, cols), lambda i: (i, 0, 0))
    blk3 = pl.BlockSpec((1, rows, 3, cols), lambda i: (i, 0, 0, 0))
    blk1 = pl.BlockSpec((1, rows, 1, cols), lambda i: (i, 0, 0, 0))
    full = lambda s: pl.BlockSpec(s, lambda i: (0,) * len(s))
    return pl.pallas_call(
        functools.partial(_eprep_body, n=n),
        grid=grid,
        in_specs=[blk, blk, blk, blk, full(le1.shape), full(e1.shape),
                  full(le2.shape), full(e2.shape)],
        out_specs=[blk3, blk1, blk1],
        out_shape=[
            jax.ShapeDtypeStruct((g, rows, 3, cols), jnp.int32),
            jax.ShapeDtypeStruct((g, rows, 1, cols), jnp.float32),
            jax.ShapeDtypeStruct((g, rows, 1, cols), jnp.float32),
        ],
    )(dst2, src2, typ2, attr2, le1, e1, le2, e2)


def _finish_body(acc_ref, den_ref, b_ref, out_ref, *, relu):
    o = acc_ref[...] / (den_ref[...] + 1e-16) + b_ref[...]
    if relu:
        o = jnp.maximum(o, 0.0)
    out_ref[...] = o


def _finish(acc, den, b, relu, bn):
    m, c = acc.shape
    grid = (m // bn,)
    return pl.pallas_call(
        functools.partial(_finish_body, relu=relu),
        grid=grid,
        in_specs=[
            pl.BlockSpec((bn, c), lambda i: (i, 0)),
            pl.BlockSpec((bn, 1), lambda i: (i, 0)),
            pl.BlockSpec((1, c), lambda i: (0, 0)),
        ],
        out_specs=pl.BlockSpec((bn, c), lambda i: (i, 0)),
        out_shape=jax.ShapeDtypeStruct((m, c), jnp.float32),
    )(acc, den, b)


def _make_sc_edge(n_chunks, tph, hp, half, c, n):
    mesh = plsc.VectorSubcoreMesh(core_axis_name="c", subcore_axis_name="s")

    @functools.partial(
        pl.kernel,
        mesh=mesh,
        compiler_params=pltpu.CompilerParams(needs_layout_passes=False),
        out_type=[
            jax.ShapeDtypeStruct((NC, hp, c), jnp.float32),
            jax.ShapeDtypeStruct((NC * hp,), jnp.float32),
        ],
        scratch_types=[
            pltpu.VMEM((3 * n,), jnp.float32),
            pltpu.VMEM((3 * n,), jnp.float32),
        ] + 2 * [
            pltpu.VMEM((3, CH), jnp.int32),
            pltpu.VMEM((1, CH), jnp.float32),
            pltpu.VMEM((CH,), jnp.float32),
            pltpu.VMEM((1, CH), jnp.int32),
            pltpu.VMEM((1, CH), jnp.int32),
            pltpu.VMEM((CH, c), jnp.float32),
        ] + [
            pltpu.VMEM_SHARED((hp, c), jnp.float32),
            pltpu.VMEM_SHARED((hp,), jnp.float32),
            pltpu.SemaphoreType.DMA,
            pltpu.SemaphoreType.DMA,
            pltpu.SemaphoreType.DMA,
            pltpu.SemaphoreType.DMA,
            pltpu.SemaphoreType.DMA,
            pltpu.SemaphoreType.DMA,
        ],
    )
    def sc_edge(xw_h, sq_h, sk_h, edata_h, aed_h, zrow_h, zden_h,
                accp_h, denp_h,
                sq_t, sk_t,
                e0, ae0, ev0, dl0, gi0, rows0,
                e1, ae1, ev1, dl1, gi1, rows1,
                acc_sh, den_sh, eds0, eds1, rs0, rs1, ss0, ss1):
        c_ax = lax.axis_index("c")
        s_ax = lax.axis_index("s")
        base = c_ax * half
        pltpu.sync_copy(sq_h, sq_t)
        pltpu.sync_copy(sk_h, sk_t)
        row0 = s_ax * tph
        pltpu.sync_copy(zrow_h, acc_sh.at[pl.ds(row0, tph)])
        for j in range(hp // DEN_CH):
            @pl.when(s_ax == j % NS)
            def _zero_den():
                pltpu.sync_copy(zden_h, den_sh.at[pl.ds(j * DEN_CH, DEN_CH)])
        plsc.subcore_barrier()

        bufs = ((e0, ae0, ev0, dl0, gi0, rows0, eds0, rs0, ss0),
                (e1, ae1, ev1, dl1, gi1, rows1, eds1, rs1, ss1))

        def sc_wait(b):
            _, _, ev_v, dl_v, _, rows_v, _, _, ssem = bufs[b]
            pltpu.make_async_copy(
                rows_v, acc_sh.at[dl_v.at[0]], ssem).wait()
            pltpu.make_async_copy(
                ev_v, den_sh.at[dl_v.at[0]], ssem).wait()

        def ed_start(ci, b):
            e_v, ae_v = bufs[b][0], bufs[b][1]
            eds = bufs[b][6]
            pltpu.async_copy(edata_h.at[s_ax, ci], e_v, eds)
            pltpu.async_copy(aed_h.at[s_ax, ci], ae_v, eds)

        def ed_wait(ci, b):
            e_v, ae_v = bufs[b][0], bufs[b][1]
            eds = bufs[b][6]
            pltpu.make_async_copy(edata_h.at[s_ax, ci], e_v, eds).wait()
            pltpu.make_async_copy(aed_h.at[s_ax, ci], ae_v, eds).wait()

        def phase_b(ci, b, wpred):
            e_v, ae_v, ev_v, dl_v, gi_v, rows_v, eds, rsem, ssem = bufs[b]
            ed_wait(ci, b)

            @pl.when(wpred)
            def _wait_prev_scatter():
                sc_wait(b)
            for g in range(CH // LANES):
                sl = pl.ds(g * LANES, LANES)
                gq16 = e_v[0, sl]
                gk16 = e_v[1, sl]
                ae = ae_v[0, sl]
                sqv = plsc.load_gather(sq_t, [gq16])
                skv = plsc.load_gather(sk_t, [gk16])
                a = sqv + skv + ae
                a = jnp.maximum(a, a * NEG_SLOPE)
                ev_v[sl] = jnp.exp(a)
                dloc = e_v[2, sl] - base
                owned = (dloc >= 0) & (dloc < half)
                dl_v[0, sl] = jnp.where(owned, dloc, half)
                gi_v[0, sl] = gk16
            return pltpu.async_copy(xw_h.at[gi_v.at[0]], rows_v, rsem)

        def drain(ci, b):
            e_v, ae_v, ev_v, dl_v, gi_v, rows_v, eds, rsem, ssem = bufs[b]

            def scale(si, cc):
                for k in range(4):
                    ei = si * 4 + k
                    idxb = jnp.full((LANES,), ei, dtype=jnp.int32)
                    evb = plsc.load_gather(ev_v, [idxb])
                    for j in range(c // LANES):
                        sl = pl.ds(j * LANES, LANES)
                        rows_v[ei, sl] = rows_v[ei, sl] * evb
                return cc

            lax.fori_loop(0, CH // 4, scale, 0)
            pltpu.async_copy(rows_v, acc_sh.at[dl_v.at[0]], ssem, add=True)
            pltpu.async_copy(ev_v, den_sh.at[dl_v.at[0]], ssem, add=True)

        ed_start(0, 0)
        ed_start(1, 1)

        def pair(p, carry):
            ci = 2 * p
            h0 = phase_b(ci, 0, p > 0)
            ed_start(jnp.minimum(ci + 2, n_chunks - 1), 0)
            h1 = phase_b(ci + 1, 1, p > 0)
            ed_start(jnp.minimum(ci + 3, n_chunks - 1), 1)
            h0.wait()
            drain(ci, 0)
            h1.wait()
            drain(ci + 1, 1)
            return carry

        lax.fori_loop(0, n_chunks // 2, pair, 0)
        ed_wait(0, 0)
        ed_wait(0, 1)
        sc_wait(0)
        sc_wait(1)
        plsc.subcore_barrier()
        pltpu.sync_copy(acc_sh.at[pl.ds(row0, tph)],
                        accp_h.at[c_ax, pl.ds(row0, tph)])
        for j in range(hp // DEN_CH):
            @pl.when(s_ax == j % NS)
            def _copy_den():
                pltpu.sync_copy(
                    den_sh.at[pl.ds(j * DEN_CH, DEN_CH)],
                    denp_h.at[pl.ds(c_ax * hp + j * DEN_CH, DEN_CH)])

    return sc_edge


def kernel(x, edge_index, edge_type, edge_attr,
           w1, q1, k1, le1, e1, b1,
           w2, q2, k2, le2, e2, b2):
    n, cin = x.shape
    e = edge_type.shape[0]
    hid = w1.shape[2]
    out_c = w2.shape[2]

    per_tile = -(-e // (NS * 2 * CH)) * 2 * CH
    e_pad = per_tile * NS
    n_chunks = per_tile // CH
    half = (n + 1) // 2
    hp = -(-(half + 1) // DEN_CH) * DEN_CH
    tph = hp // NS

    src = edge_index[0]
    dst = edge_index[1]
    typ = edge_type
    attr = edge_attr.reshape(e)
    if e_pad != e:
        pad = e_pad - e
        src = jnp.pad(src, (0, pad))
        dst = jnp.pad(dst, (0, pad), constant_values=n)
        typ = jnp.pad(typ, (0, pad))
        attr = jnp.pad(attr, (0, pad), constant_values=-1e30)

    dst2 = dst.reshape(NS, n_chunks, CH)
    src2 = src.reshape(NS, n_chunks, CH)
    typ2 = typ.reshape(NS, n_chunks, CH)
    attr2 = attr.reshape(NS, n_chunks, CH)

    edata, aed1, aed2 = _eprep(dst2, src2, typ2, attr2,
                               le1, e1, le2, e2, n)

    zrow = jnp.zeros((tph, hid), jnp.float32)
    zden = jnp.zeros((DEN_CH,), jnp.float32)

    sc_edge = _make_sc_edge(n_chunks, tph, hp, half, hid, n)

    def assemble(accp, denp):
        acc = accp[:, :half].reshape(2 * half, hid)[:n]
        den = denp.reshape(NC, hp)[:, :half].reshape(2 * half)[:n]
        return acc, den.reshape(n, 1)

    xw3, sq3, sk3 = _prep(x, w1, q1, k1, bn=2000)
    xw = xw3.reshape(3 * n, hid)
    sq = sq3.reshape(3 * n)
    sk = sk3.reshape(3 * n)
    accp, denp = sc_edge(xw, sq, sk, edata, aed1, zrow, zden)
    acc1, den1 = assemble(accp, denp)
    h = _finish(acc1, den1, b1.reshape(1, hid), relu=True, bn=2000)

    xw3b, sq3b, sk3b = _prep(h, w2, q2, k2, bn=2000)
    xwb = xw3b.reshape(3 * n, out_c)
    sqb = sq3b.reshape(3 * n)
    skb = sk3b.reshape(3 * n)
    accp2, denp2 = sc_edge(xwb, sqb, skb, edata, aed2, zrow, zden)
    acc2, den2 = assemble(accp2, denp2)
    out = _finish(acc2, den2, b2.reshape(1, out_c), relu=False, bn=2000)
    return out


# speedup vs baseline: 1.1034x; 1.1034x over previous
import functools

import jax
import jax.numpy as jnp
from jax import lax
from jax.experimental import pallas as pl
from jax.experimental.pallas import tpu as pltpu
from jax.experimental.pallas import tpu_sc as plsc

NEG_SLOPE = 0.2
NC = 2
NS = 16
NW = NC * NS
CH = 64
LANES = 16
DEN_CH = 512


def _prep_body(x_ref, w_ref, q_ref, k_ref, xw_ref, sq_ref, sk_ref):
    xb = x_ref[...]
    o = jnp.dot(xb, w_ref[0], preferred_element_type=jnp.float32)
    xw_ref[0] = o
    sq_ref[0] = jnp.dot(o, q_ref[...], preferred_element_type=jnp.float32)
    sk_ref[0] = jnp.dot(o, k_ref[...], preferred_element_type=jnp.float32)


def _prep(x, w, q, k, bn):
    n, c = x.shape
    r = w.shape[0]
    grid = (r, n // bn)
    return pl.pallas_call(
        _prep_body,
        grid=grid,
        in_specs=[
            pl.BlockSpec((bn, c), lambda ri, i: (i, 0)),
            pl.BlockSpec((1, c, w.shape[2]), lambda ri, i: (ri, 0, 0)),
            pl.BlockSpec(q.shape, lambda ri, i: (0, 0)),
            pl.BlockSpec(k.shape, lambda ri, i: (0, 0)),
        ],
        out_specs=[
            pl.BlockSpec((1, bn, w.shape[2]), lambda ri, i: (ri, i, 0)),
            pl.BlockSpec((1, bn, 1), lambda ri, i: (ri, i, 0)),
            pl.BlockSpec((1, bn, 1), lambda ri, i: (ri, i, 0)),
        ],
        out_shape=[
            jax.ShapeDtypeStruct((r, n, w.shape[2]), jnp.float32),
            jax.ShapeDtypeStruct((r, n, 1), jnp.float32),
            jax.ShapeDtypeStruct((r, n, 1), jnp.float32),
        ],
    )(x, w, q, k)


def _eprep_body(dst_ref, src_ref, typ_ref, attr_ref, le1_ref, e1_ref,
                le2_ref, e2_ref, ed_ref, ae1_ref, ae2_ref, *, n):
    t = typ_ref[0]
    ed_ref[0, :, 0, :] = t * n + dst_ref[0]
    ed_ref[0, :, 1, :] = t * n + src_ref[0]
    ed_ref[0, :, 2, :] = dst_ref[0]
    c1 = jnp.sum(jnp.dot(le1_ref[...], e1_ref[...],
                         preferred_element_type=jnp.float32))
    c2 = jnp.sum(jnp.dot(le2_ref[...], e2_ref[...],
                         preferred_element_type=jnp.float32))
    a = attr_ref[0]
    ae1_ref[0, :, 0, :] = a * c1
    ae2_ref[0, :, 0, :] = a * c2


def _eprep(dst2, src2, typ2, attr2, le1, e1, le2, e2, n):
    g, rows, cols = dst2.shape
    grid = (g,)
    blk = pl.BlockSpec((1, rows, cols), lambda i: (i, 0, 0))
    blk3 = pl.BlockSpec((1, rows, 3, cols), lambda i: (i, 0, 0, 0))
    blk1 = pl.BlockSpec((1, rows, 1, cols), lambda i: (i, 0, 0, 0))
    full = lambda s: pl.BlockSpec(s, lambda i: (0,) * len(s))
    return pl.pallas_call(
        functools.partial(_eprep_body, n=n),
        grid=grid,
        in_specs=[blk, blk, blk, blk, full(le1.shape), full(e1.shape),
                  full(le2.shape), full(e2.shape)],
        out_specs=[blk3, blk1, blk1],
        out_shape=[
            jax.ShapeDtypeStruct((g, rows, 3, cols), jnp.int32),
            jax.ShapeDtypeStruct((g, rows, 1, cols), jnp.float32),
            jax.ShapeDtypeStruct((g, rows, 1, cols), jnp.float32),
        ],
    )(dst2, src2, typ2, attr2, le1, e1, le2, e2)


def _finish_body(acc_ref, den_ref, b_ref, out_ref, *, relu):
    o = acc_ref[...] / (den_ref[...] + 1e-16) + b_ref[...]
    if relu:
        o = jnp.maximum(o, 0.0)
    out_ref[...] = o


def _finish(acc, den, b, relu, bn):
    m, c = acc.shape
    grid = (m // bn,)
    return pl.pallas_call(
        functools.partial(_finish_body, relu=relu),
        grid=grid,
        in_specs=[
            pl.BlockSpec((bn, c), lambda i: (i, 0)),
            pl.BlockSpec((bn, 1), lambda i: (i, 0)),
            pl.BlockSpec((1, c), lambda i: (0, 0)),
        ],
        out_specs=pl.BlockSpec((bn, c), lambda i: (i, 0)),
        out_shape=jax.ShapeDtypeStruct((m, c), jnp.float32),
    )(acc, den, b)


def _make_sc_edge(n_chunks, tph, hp, half, c, n):
    mesh = plsc.VectorSubcoreMesh(core_axis_name="c", subcore_axis_name="s")

    @functools.partial(
        pl.kernel,
        mesh=mesh,
        compiler_params=pltpu.CompilerParams(needs_layout_passes=False),
        out_type=[
            jax.ShapeDtypeStruct((NC, hp, c), jnp.float32),
            jax.ShapeDtypeStruct((NC * hp,), jnp.float32),
        ],
        scratch_types=[
            pltpu.VMEM((3 * n,), jnp.float32),
            pltpu.VMEM((3 * n,), jnp.float32),
        ] + 2 * [
            pltpu.VMEM((3, CH), jnp.int32),
            pltpu.VMEM((1, CH), jnp.float32),
            pltpu.VMEM((CH,), jnp.float32),
            pltpu.VMEM((1, CH), jnp.int32),
            pltpu.VMEM((1, CH), jnp.int32),
            pltpu.VMEM((CH, c), jnp.float32),
        ] + [
            pltpu.VMEM_SHARED((hp, c), jnp.float32),
            pltpu.VMEM_SHARED((hp,), jnp.float32),
            pltpu.SemaphoreType.DMA,
            pltpu.SemaphoreType.DMA,
            pltpu.SemaphoreType.DMA,
            pltpu.SemaphoreType.DMA,
            pltpu.SemaphoreType.DMA,
            pltpu.SemaphoreType.DMA,
        ],
    )
    def sc_edge(xw_h, sq_h, sk_h, edata_h, aed_h, zrow_h, zden_h,
                accp_h, denp_h,
                sq_t, sk_t,
                e0, ae0, ev0, dl0, gi0, rows0,
                e1, ae1, ev1, dl1, gi1, rows1,
                acc_sh, den_sh, eds0, eds1, rs0, rs1, ss0, ss1):
        c_ax = lax.axis_index("c")
        s_ax = lax.axis_index("s")
        base = c_ax * half
        pltpu.sync_copy(sq_h, sq_t)
        pltpu.sync_copy(sk_h, sk_t)
        row0 = s_ax * tph
        pltpu.sync_copy(zrow_h, acc_sh.at[pl.ds(row0, tph)])
        for j in range(hp // DEN_CH):
            @pl.when(s_ax == j % NS)
            def _zero_den():
                pltpu.sync_copy(zden_h, den_sh.at[pl.ds(j * DEN_CH, DEN_CH)])
        plsc.subcore_barrier()

        bufs = ((e0, ae0, ev0, dl0, gi0, rows0, eds0, rs0, ss0),
                (e1, ae1, ev1, dl1, gi1, rows1, eds1, rs1, ss1))

        def sc_wait(b):
            _, _, ev_v, dl_v, _, rows_v, _, _, ssem = bufs[b]
            pltpu.make_async_copy(
                rows_v, acc_sh.at[dl_v.at[0]], ssem).wait()
            pltpu.make_async_copy(
                ev_v, den_sh.at[dl_v.at[0]], ssem).wait()

        def ed_start(ci, b):
            e_v, ae_v = bufs[b][0], bufs[b][1]
            eds = bufs[b][6]
            pltpu.async_copy(edata_h.at[s_ax, ci], e_v, eds)
            pltpu.async_copy(aed_h.at[s_ax, ci], ae_v, eds)

        def ed_wait(ci, b):
            e_v, ae_v = bufs[b][0], bufs[b][1]
            eds = bufs[b][6]
            pltpu.make_async_copy(edata_h.at[s_ax, ci], e_v, eds).wait()
            pltpu.make_async_copy(aed_h.at[s_ax, ci], ae_v, eds).wait()

        def phase_b(ci, b, wpred):
            e_v, ae_v, ev_v, dl_v, gi_v, rows_v, eds, rsem, ssem = bufs[b]
            ed_wait(ci, b)

            @pl.when(wpred)
            def _wait_prev_scatter():
                sc_wait(b)
            h = pltpu.async_copy(xw_h.at[e_v.at[1]], rows_v, rsem)
            for g in range(CH // LANES):
                sl = pl.ds(g * LANES, LANES)
                gq16 = e_v[0, sl]
                gk16 = e_v[1, sl]
                ae = ae_v[0, sl]
                sqv = plsc.load_gather(sq_t, [gq16])
                skv = plsc.load_gather(sk_t, [gk16])
                a = sqv + skv + ae
                a = jnp.maximum(a, a * NEG_SLOPE)
                ev_v[sl] = jnp.exp(a)
                dloc = e_v[2, sl] - base
                owned = (dloc >= 0) & (dloc < half)
                dl_v[0, sl] = jnp.where(owned, dloc, half)
            return h

        def drain(ci, b):
            e_v, ae_v, ev_v, dl_v, gi_v, rows_v, eds, rsem, ssem = bufs[b]

            def scale(si, cc):
                for k in range(4):
                    ei = si * 4 + k
                    idxb = jnp.full((LANES,), ei, dtype=jnp.int32)
                    evb = plsc.load_gather(ev_v, [idxb])
                    for j in range(c // LANES):
                        sl = pl.ds(j * LANES, LANES)
                        rows_v[ei, sl] = rows_v[ei, sl] * evb
                return cc

            lax.fori_loop(0, CH // 4, scale, 0)
            pltpu.async_copy(rows_v, acc_sh.at[dl_v.at[0]], ssem, add=True)
            pltpu.async_copy(ev_v, den_sh.at[dl_v.at[0]], ssem, add=True)

        ed_start(0, 0)
        ed_start(1, 1)

        def pair(p, carry):
            ci = 2 * p
            h0 = phase_b(ci, 0, p > 0)
            h1 = phase_b(ci + 1, 1, p > 0)
            h0.wait()
            ed_start(jnp.minimum(ci + 2, n_chunks - 1), 0)
            drain(ci, 0)
            h1.wait()
            ed_start(jnp.minimum(ci + 3, n_chunks - 1), 1)
            drain(ci + 1, 1)
            return carry

        lax.fori_loop(0, n_chunks // 2, pair, 0)
        ed_wait(0, 0)
        ed_wait(0, 1)
        sc_wait(0)
        sc_wait(1)
        plsc.subcore_barrier()
        pltpu.sync_copy(acc_sh.at[pl.ds(row0, tph)],
                        accp_h.at[c_ax, pl.ds(row0, tph)])
        for j in range(hp // DEN_CH):
            @pl.when(s_ax == j % NS)
            def _copy_den():
                pltpu.sync_copy(
                    den_sh.at[pl.ds(j * DEN_CH, DEN_CH)],
                    denp_h.at[pl.ds(c_ax * hp + j * DEN_CH, DEN_CH)])

    return sc_edge


def kernel(x, edge_index, edge_type, edge_attr,
           w1, q1, k1, le1, e1, b1,
           w2, q2, k2, le2, e2, b2):
    n, cin = x.shape
    e = edge_type.shape[0]
    hid = w1.shape[2]
    out_c = w2.shape[2]

    per_tile = -(-e // (NS * 2 * CH)) * 2 * CH
    e_pad = per_tile * NS
    n_chunks = per_tile // CH
    half = (n + 1) // 2
    hp = -(-(half + 1) // DEN_CH) * DEN_CH
    tph = hp // NS

    src = edge_index[0]
    dst = edge_index[1]
    typ = edge_type
    attr = edge_attr.reshape(e)
    if e_pad != e:
        pad = e_pad - e
        src = jnp.pad(src, (0, pad))
        dst = jnp.pad(dst, (0, pad), constant_values=n)
        typ = jnp.pad(typ, (0, pad))
        attr = jnp.pad(attr, (0, pad), constant_values=-1e30)

    dst2 = dst.reshape(NS, n_chunks, CH)
    src2 = src.reshape(NS, n_chunks, CH)
    typ2 = typ.reshape(NS, n_chunks, CH)
    attr2 = attr.reshape(NS, n_chunks, CH)

    edata, aed1, aed2 = _eprep(dst2, src2, typ2, attr2,
                               le1, e1, le2, e2, n)

    zrow = jnp.zeros((tph, hid), jnp.float32)
    zden = jnp.zeros((DEN_CH,), jnp.float32)

    sc_edge = _make_sc_edge(n_chunks, tph, hp, half, hid, n)

    def assemble(accp, denp):
        acc = accp[:, :half].reshape(2 * half, hid)[:n]
        den = denp.reshape(NC, hp)[:, :half].reshape(2 * half)[:n]
        return acc, den.reshape(n, 1)

    xw3, sq3, sk3 = _prep(x, w1, q1, k1, bn=2000)
    xw = xw3.reshape(3 * n, hid)
    sq = sq3.reshape(3 * n)
    sk = sk3.reshape(3 * n)
    accp, denp = sc_edge(xw, sq, sk, edata, aed1, zrow, zden)
    acc1, den1 = assemble(accp, denp)
    h = _finish(acc1, den1, b1.reshape(1, hid), relu=True, bn=2000)

    xw3b, sq3b, sk3b = _prep(h, w2, q2, k2, bn=2000)
    xwb = xw3b.reshape(3 * n, out_c)
    sqb = sq3b.reshape(3 * n)
    skb = sk3b.reshape(3 * n)
    accp2, denp2 = sc_edge(xwb, sqb, skb, edata, aed2, zrow, zden)
    acc2, den2 = assemble(accp2, denp2)
    out = _finish(acc2, den2, b2.reshape(1, out_c), relu=False, bn=2000)
    return out


# speedup vs baseline: 1.1094x; 1.0054x over previous
import functools

import jax
import jax.numpy as jnp
from jax import lax
from jax.experimental import pallas as pl
from jax.experimental.pallas import tpu as pltpu
from jax.experimental.pallas import tpu_sc as plsc

NEG_SLOPE = 0.2
NC = 2
NS = 16
NW = NC * NS
CH = 64
LANES = 16
DEN_CH = 512


def _prep_body(x_ref, w_ref, q_ref, k_ref, xw_ref, sq_ref, sk_ref):
    xb = x_ref[...]
    o = jnp.dot(xb, w_ref[0], preferred_element_type=jnp.float32)
    xw_ref[0] = o
    sq_ref[0] = jnp.dot(o, q_ref[...], preferred_element_type=jnp.float32)
    sk_ref[0] = jnp.dot(o, k_ref[...], preferred_element_type=jnp.float32)


def _prep(x, w, q, k, bn):
    n, c = x.shape
    r = w.shape[0]
    grid = (r, n // bn)
    return pl.pallas_call(
        _prep_body,
        grid=grid,
        in_specs=[
            pl.BlockSpec((bn, c), lambda ri, i: (i, 0)),
            pl.BlockSpec((1, c, w.shape[2]), lambda ri, i: (ri, 0, 0)),
            pl.BlockSpec(q.shape, lambda ri, i: (0, 0)),
            pl.BlockSpec(k.shape, lambda ri, i: (0, 0)),
        ],
        out_specs=[
            pl.BlockSpec((1, bn, w.shape[2]), lambda ri, i: (ri, i, 0)),
            pl.BlockSpec((1, bn, 1), lambda ri, i: (ri, i, 0)),
            pl.BlockSpec((1, bn, 1), lambda ri, i: (ri, i, 0)),
        ],
        out_shape=[
            jax.ShapeDtypeStruct((r, n, w.shape[2]), jnp.float32),
            jax.ShapeDtypeStruct((r, n, 1), jnp.float32),
            jax.ShapeDtypeStruct((r, n, 1), jnp.float32),
        ],
    )(x, w, q, k)


def _eprep_body(dst_ref, src_ref, typ_ref, attr_ref, le1_ref, e1_ref,
                le2_ref, e2_ref, ed_ref, ae1_ref, ae2_ref, *, n):
    t = typ_ref[0]
    ed_ref[0, :, 0, :] = t * n + dst_ref[0]
    ed_ref[0, :, 1, :] = t * n + src_ref[0]
    ed_ref[0, :, 2, :] = dst_ref[0]
    c1 = jnp.sum(jnp.dot(le1_ref[...], e1_ref[...],
                         preferred_element_type=jnp.float32))
    c2 = jnp.sum(jnp.dot(le2_ref[...], e2_ref[...],
                         preferred_element_type=jnp.float32))
    a = attr_ref[0]
    ae1_ref[0, :, 0, :] = a * c1
    ae2_ref[0, :, 0, :] = a * c2


def _eprep(dst2, src2, typ2, attr2, le1, e1, le2, e2, n):
    g, rows, cols = dst2.shape
    grid = (g,)
    blk = pl.BlockSpec((1, rows, cols), lambda i: (i, 0, 0))
    blk3 = pl.BlockSpec((1, rows, 3, cols), lambda i: (i, 0, 0, 0))
    blk1 = pl.BlockSpec((1, rows, 1, cols), lambda i: (i, 0, 0, 0))
    full = lambda s: pl.BlockSpec(s, lambda i: (0,) * len(s))
    return pl.pallas_call(
        functools.partial(_eprep_body, n=n),
        grid=grid,
        in_specs=[blk, blk, blk, blk, full(le1.shape), full(e1.shape),
                  full(le2.shape), full(e2.shape)],
        out_specs=[blk3, blk1, blk1],
        out_shape=[
            jax.ShapeDtypeStruct((g, rows, 3, cols), jnp.int32),
            jax.ShapeDtypeStruct((g, rows, 1, cols), jnp.float32),
            jax.ShapeDtypeStruct((g, rows, 1, cols), jnp.float32),
        ],
    )(dst2, src2, typ2, attr2, le1, e1, le2, e2)


def _finish_body(acc_ref, den_ref, b_ref, out_ref, *, relu):
    o = acc_ref[0] / (den_ref[...] + 1e-16) + b_ref[...]
    if relu:
        o = jnp.maximum(o, 0.0)
    out_ref[...] = o


def _finish(accp, den, b, relu, half, n):
    c = accp.shape[2]
    bn = next(d for d in (2000, 1000, 500, 200, 100, 40, 8)
              if half % d == 0 and d % 8 == 0)
    bpc = half // bn
    grid = (n // bn,)
    return pl.pallas_call(
        functools.partial(_finish_body, relu=relu),
        grid=grid,
        in_specs=[
            pl.BlockSpec((1, bn, c), lambda i: (i // bpc, i % bpc, 0)),
            pl.BlockSpec((bn, 1), lambda i: (i, 0)),
            pl.BlockSpec((1, c), lambda i: (0, 0)),
        ],
        out_specs=pl.BlockSpec((bn, c), lambda i: (i, 0)),
        out_shape=jax.ShapeDtypeStruct((n, c), jnp.float32),
    )(accp, den, b)


def _make_sc_edge(n_chunks, tph, hp, half, c, n):
    mesh = plsc.VectorSubcoreMesh(core_axis_name="c", subcore_axis_name="s")

    @functools.partial(
        pl.kernel,
        mesh=mesh,
        compiler_params=pltpu.CompilerParams(needs_layout_passes=False),
        out_type=[
            jax.ShapeDtypeStruct((NC, hp, c), jnp.float32),
            jax.ShapeDtypeStruct((NC * hp,), jnp.float32),
        ],
        scratch_types=[
            pltpu.VMEM((3 * n,), jnp.float32),
            pltpu.VMEM((3 * n,), jnp.float32),
        ] + 2 * [
            pltpu.VMEM((3, CH), jnp.int32),
            pltpu.VMEM((1, CH), jnp.float32),
            pltpu.VMEM((CH,), jnp.float32),
            pltpu.VMEM((1, CH), jnp.int32),
            pltpu.VMEM((1, CH), jnp.int32),
            pltpu.VMEM((CH, c), jnp.float32),
        ] + [
            pltpu.VMEM_SHARED((hp, c), jnp.float32),
            pltpu.VMEM_SHARED((hp,), jnp.float32),
            pltpu.SemaphoreType.DMA,
            pltpu.SemaphoreType.DMA,
            pltpu.SemaphoreType.DMA,
            pltpu.SemaphoreType.DMA,
            pltpu.SemaphoreType.DMA,
            pltpu.SemaphoreType.DMA,
        ],
    )
    def sc_edge(xw_h, sq_h, sk_h, edata_h, aed_h, zrow_h, zden_h,
                accp_h, denp_h,
                sq_t, sk_t,
                e0, ae0, ev0, dl0, gi0, rows0,
                e1, ae1, ev1, dl1, gi1, rows1,
                acc_sh, den_sh, eds0, eds1, rs0, rs1, ss0, ss1):
        c_ax = lax.axis_index("c")
        s_ax = lax.axis_index("s")
        base = c_ax * half
        pltpu.sync_copy(sq_h, sq_t)
        pltpu.sync_copy(sk_h, sk_t)
        row0 = s_ax * tph
        pltpu.sync_copy(zrow_h, acc_sh.at[pl.ds(row0, tph)])
        for j in range(hp // DEN_CH):
            @pl.when(s_ax == j % NS)
            def _zero_den():
                pltpu.sync_copy(zden_h, den_sh.at[pl.ds(j * DEN_CH, DEN_CH)])
        plsc.subcore_barrier()

        bufs = ((e0, ae0, ev0, dl0, gi0, rows0, eds0, rs0, ss0),
                (e1, ae1, ev1, dl1, gi1, rows1, eds1, rs1, ss1))

        def sc_wait(b):
            _, _, ev_v, dl_v, _, rows_v, _, _, ssem = bufs[b]
            pltpu.make_async_copy(
                rows_v, acc_sh.at[dl_v.at[0]], ssem).wait()
            pltpu.make_async_copy(
                ev_v, den_sh.at[dl_v.at[0]], ssem).wait()

        def ed_start(ci, b):
            e_v, ae_v = bufs[b][0], bufs[b][1]
            eds = bufs[b][6]
            pltpu.async_copy(edata_h.at[s_ax, ci], e_v, eds)
            pltpu.async_copy(aed_h.at[s_ax, ci], ae_v, eds)

        def ed_wait(ci, b):
            e_v, ae_v = bufs[b][0], bufs[b][1]
            eds = bufs[b][6]
            pltpu.make_async_copy(edata_h.at[s_ax, ci], e_v, eds).wait()
            pltpu.make_async_copy(aed_h.at[s_ax, ci], ae_v, eds).wait()

        def phase_b(ci, b, wpred):
            e_v, ae_v, ev_v, dl_v, gi_v, rows_v, eds, rsem, ssem = bufs[b]
            ed_wait(ci, b)

            @pl.when(wpred)
            def _wait_prev_scatter():
                sc_wait(b)
            h = pltpu.async_copy(xw_h.at[e_v.at[1]], rows_v, rsem)
            for g in range(CH // LANES):
                sl = pl.ds(g * LANES, LANES)
                gq16 = e_v[0, sl]
                gk16 = e_v[1, sl]
                ae = ae_v[0, sl]
                sqv = plsc.load_gather(sq_t, [gq16])
                skv = plsc.load_gather(sk_t, [gk16])
                a = sqv + skv + ae
                a = jnp.maximum(a, a * NEG_SLOPE)
                ev_v[sl] = jnp.exp(a)
                dloc = e_v[2, sl] - base
                owned = (dloc >= 0) & (dloc < half)
                dl_v[0, sl] = jnp.where(owned, dloc, half)
            return h

        def drain(ci, b):
            e_v, ae_v, ev_v, dl_v, gi_v, rows_v, eds, rsem, ssem = bufs[b]

            def scale(si, cc):
                for k in range(8):
                    ei = si * 8 + k
                    idxb = jnp.full((LANES,), ei, dtype=jnp.int32)
                    evb = plsc.load_gather(ev_v, [idxb])
                    for j in range(c // LANES):
                        sl = pl.ds(j * LANES, LANES)
                        rows_v[ei, sl] = rows_v[ei, sl] * evb
                return cc

            lax.fori_loop(0, CH // 8, scale, 0)
            pltpu.async_copy(rows_v, acc_sh.at[dl_v.at[0]], ssem, add=True)
            pltpu.async_copy(ev_v, den_sh.at[dl_v.at[0]], ssem, add=True)

        ed_start(0, 0)
        ed_start(1, 1)

        def pair(p, carry):
            ci = 2 * p
            h0 = phase_b(ci, 0, p > 0)
            h1 = phase_b(ci + 1, 1, p > 0)
            h0.wait()
            ed_start(jnp.minimum(ci + 2, n_chunks - 1), 0)
            drain(ci, 0)
            h1.wait()
            ed_start(jnp.minimum(ci + 3, n_chunks - 1), 1)
            drain(ci + 1, 1)
            return carry

        lax.fori_loop(0, n_chunks // 2, pair, 0)
        ed_wait(0, 0)
        ed_wait(0, 1)
        sc_wait(0)
        sc_wait(1)
        plsc.subcore_barrier()
        pltpu.sync_copy(acc_sh.at[pl.ds(row0, tph)],
                        accp_h.at[c_ax, pl.ds(row0, tph)])
        for j in range(hp // DEN_CH):
            @pl.when(s_ax == j % NS)
            def _copy_den():
                pltpu.sync_copy(
                    den_sh.at[pl.ds(j * DEN_CH, DEN_CH)],
                    denp_h.at[pl.ds(c_ax * hp + j * DEN_CH, DEN_CH)])

    return sc_edge


def kernel(x, edge_index, edge_type, edge_attr,
           w1, q1, k1, le1, e1, b1,
           w2, q2, k2, le2, e2, b2):
    n, cin = x.shape
    e = edge_type.shape[0]
    hid = w1.shape[2]
    out_c = w2.shape[2]

    per_tile = -(-e // (NS * 2 * CH)) * 2 * CH
    e_pad = per_tile * NS
    n_chunks = per_tile // CH
    half = (n + 1) // 2
    hp = -(-(half + 1) // DEN_CH) * DEN_CH
    tph = hp // NS

    src = edge_index[0]
    dst = edge_index[1]
    typ = edge_type
    attr = edge_attr.reshape(e)
    if e_pad != e:
        pad = e_pad - e
        src = jnp.pad(src, (0, pad))
        dst = jnp.pad(dst, (0, pad), constant_values=n)
        typ = jnp.pad(typ, (0, pad))
        attr = jnp.pad(attr, (0, pad), constant_values=-1e30)

    dst2 = dst.reshape(NS, n_chunks, CH)
    src2 = src.reshape(NS, n_chunks, CH)
    typ2 = typ.reshape(NS, n_chunks, CH)
    attr2 = attr.reshape(NS, n_chunks, CH)

    edata, aed1, aed2 = _eprep(dst2, src2, typ2, attr2,
                               le1, e1, le2, e2, n)

    zrow = jnp.zeros((tph, hid), jnp.float32)
    zden = jnp.zeros((DEN_CH,), jnp.float32)

    sc_edge = _make_sc_edge(n_chunks, tph, hp, half, hid, n)

    def den_col(denp):
        return denp.reshape(NC, hp)[:, :half].reshape(2 * half)[:n, None]

    xw3, sq3, sk3 = _prep(x, w1, q1, k1, bn=2000)
    xw = xw3.reshape(3 * n, hid)
    sq = sq3.reshape(3 * n)
    sk = sk3.reshape(3 * n)
    accp, denp = sc_edge(xw, sq, sk, edata, aed1, zrow, zden)
    h = _finish(accp, den_col(denp), b1.reshape(1, hid), relu=True,
                half=half, n=n)

    xw3b, sq3b, sk3b = _prep(h, w2, q2, k2, bn=2000)
    xwb = xw3b.reshape(3 * n, out_c)
    sqb = sq3b.reshape(3 * n)
    skb = sk3b.reshape(3 * n)
    accp2, denp2 = sc_edge(xwb, sqb, skb, edata, aed2, zrow, zden)
    out = _finish(accp2, den_col(denp2), b2.reshape(1, out_c), relu=False,
                  half=half, n=n)
    return out
